# baseline (speedup 1.0000x reference)
import functools

import jax
import jax.numpy as jnp
from jax import lax
from jax.experimental import pallas as pl
from jax.experimental.pallas import tpu as pltpu
from jax.experimental.pallas import tpu_sc as plsc

NC = 2
NS = 16
L = 16
NW = NC * NS

B = 2048


def _build_sc_call(NT, V):
    per_w = NT // NW
    nchunk = per_w // B
    mesh = plsc.VectorSubcoreMesh(core_axis_name="c", subcore_axis_name="s")

    @functools.partial(
        pl.kernel,
        mesh=mesh,
        compiler_params=pltpu.CompilerParams(needs_layout_passes=False),
        out_type=[
            jax.ShapeDtypeStruct((NT,), jnp.float32),
            jax.ShapeDtypeStruct((NT * 6,), jnp.float32),
        ],
        scratch_types=[
            pltpu.VMEM((3 * B,), jnp.int32),
            pltpu.VMEM((3 * B,), jnp.float32),
            pltpu.VMEM((3 * B,), jnp.float32),
            pltpu.VMEM((B,), jnp.float32),
            pltpu.VMEM((6 * B,), jnp.float32),
            pltpu.SemaphoreType.DMA,
        ],
    )
    def sck(nodex_hbm, nodey_hbm, elemf_hbm, area_hbm, dl_hbm, idx_v, xs_v,
            ys_v, area_v, dl_v, sem):
        wid = lax.axis_index("s") * NC + lax.axis_index("c")
        lane = lax.iota(jnp.int32, 16)

        def chunk_body(t, _):
            base = wid * per_w + t * B
            pltpu.sync_copy(elemf_hbm.at[pl.ds(base * 3, 3 * B)], idx_v)
            cx = pltpu.async_copy(nodex_hbm.at[idx_v], xs_v, sem)
            cy = pltpu.async_copy(nodey_hbm.at[idx_v], ys_v, sem)
            cx.wait()
            cy.wait()

            def g_body(g, _):
                e_i = lane + g * 16
                e3 = e_i * 3
                f = e_i * 6
                p0x = plsc.load_gather(xs_v, [e3])
                p0y = plsc.load_gather(ys_v, [e3])
                p1x = plsc.load_gather(xs_v, [e3 + 1])
                p1y = plsc.load_gather(ys_v, [e3 + 1])
                p2x = plsc.load_gather(xs_v, [e3 + 2])
                p2y = plsc.load_gather(ys_v, [e3 + 2])
                ve1x = p2x - p1x
                ve1y = p2y - p1y
                ve2x = p0x - p2x
                ve2y = p0y - p2y
                ve3x = p1x - p0x
                ve3y = p1y - p0y
                t2 = ve3y * ve2x - ve3x * ve2y
                ar = 0.5 * t2
                inv = 1.0 / t2
                ninv = -inv
                area_v[pl.ds(g * 16, 16)] = ar
                plsc.store_scatter(dl_v, [f], ve1y * ninv)
                plsc.store_scatter(dl_v, [f + 1], ve2y * ninv)
                plsc.store_scatter(dl_v, [f + 2], ve3y * ninv)
                plsc.store_scatter(dl_v, [f + 3], ve1x * inv)
                plsc.store_scatter(dl_v, [f + 4], ve2x * inv)
                plsc.store_scatter(dl_v, [f + 5], ve3x * inv)
                return 0

            lax.fori_loop(0, B // 16, g_body, 0)
            pltpu.sync_copy(area_v, area_hbm.at[pl.ds(base, B)])
            pltpu.sync_copy(dl_v, dl_hbm.at[pl.ds(base * 6, 6 * B)])
            return 0

        lax.fori_loop(0, nchunk, chunk_body, 0)

    return sck


def kernel(node, elem, x):
    NT = elem.shape[0]
    V = node.shape[0]
    assert NT % (NW * B) == 0
    elemf = elem.astype(jnp.int32).reshape(-1)
    node_t = node.T
    area, dl = _build_sc_call(NT, V)(node_t[0], node_t[1], elemf)
    return area, dl.reshape(NT, 2, 3)


_ = pl.pallas_call
